# baseline (speedup 1.0000x reference)
import functools

import jax
import jax.numpy as jnp
from jax import lax
from jax.experimental import pallas as pl
from jax.experimental.pallas import tpu as pltpu
from jax.experimental.pallas import tpu_sc as plsc

NC = 2
NS = 16
NW = NC * NS

CS = 128
SB = 16
NGRAPHS = 64


def _sc_mesh():
    return plsc.VectorSubcoreMesh(core_axis_name="c", subcore_axis_name="s")


def _make_deg_kernel(npad, ch, rpt):
    @functools.partial(
        pl.kernel,
        out_type=jax.ShapeDtypeStruct((NC, npad, 16), jnp.float32),
        mesh=_sc_mesh(),
        scratch_types=[
            pltpu.VMEM((ch, CS), jnp.int32),
            pltpu.VMEM((CS, 16), jnp.float32),
            pltpu.VMEM_SHARED((npad, 16), jnp.float32),
        ],
    )
    def deg_kernel(dst_hbm, zeros_hbm, out_hbm, dst_v, ones_v, dacc):
        c = lax.axis_index("c")
        s = lax.axis_index("s")
        wid = s * NC + c
        pltpu.sync_copy(dst_hbm.at[wid], dst_v)

        def _init(i, carry):
            ones_v[i, :] = jnp.ones((16,), jnp.float32)
            return carry

        lax.fori_loop(0, CS, _init, 0)
        pltpu.sync_copy(zeros_hbm, dacc.at[pl.ds(s * rpt, rpt)])
        plsc.subcore_barrier()

        def _body(j, carry):
            pltpu.sync_copy(ones_v, dacc.at[dst_v.at[j]], add=True)
            return carry

        lax.fori_loop(0, ch, _body, 0)
        plsc.subcore_barrier()
        pltpu.sync_copy(dacc.at[pl.ds(s * rpt, rpt)],
                        out_hbm.at[c, pl.ds(s * rpt, rpt)])

    return deg_kernel


def _make_scatter_kernel(npad, d, ch, rpt):
    assert ch % SB == 0 and SB % 2 == 0
    nsb = ch // SB

    @functools.partial(
        pl.kernel,
        out_type=jax.ShapeDtypeStruct((NC, npad, d), jnp.float32),
        mesh=_sc_mesh(),
        scratch_types=[
            pltpu.VMEM((SB, CS), jnp.int32),
            pltpu.VMEM((ch, CS), jnp.int32),
            pltpu.VMEM((CS, d), jnp.float32),
            pltpu.VMEM((CS, d), jnp.float32),
            pltpu.VMEM_SHARED((npad, d), jnp.float32),
            pltpu.SemaphoreType.DMA,
            pltpu.SemaphoreType.DMA,
        ],
    )
    def scatter_kernel(y_hbm, src_hbm, dst_hbm, zeros_hbm, out_hbm,
                       src_v, dst_v, rows0, rows1, acc, sem0, sem1):
        c = lax.axis_index("c")
        s = lax.axis_index("s")
        wid = s * NC + c
        pltpu.sync_copy(dst_hbm.at[wid], dst_v)
        pltpu.sync_copy(zeros_hbm, acc.at[pl.ds(s * rpt, rpt)])
        plsc.subcore_barrier()

        def _scatter(jj, buf):
            pltpu.sync_copy(buf, acc.at[dst_v.at[jj]], add=True)

        def _superchunk(sb, carry):
            base = sb * SB
            pltpu.sync_copy(src_hbm.at[wid, pl.ds(base, SB)], src_v)

            def _pair(i, inner):
                j0 = 2 * i
                h0 = pltpu.async_copy(y_hbm.at[src_v.at[j0]], rows0, sem0)
                h1 = pltpu.async_copy(y_hbm.at[src_v.at[j0 + 1]], rows1, sem1)
                h0.wait()
                _scatter(base + j0, rows0)
                h1.wait()
                _scatter(base + j0 + 1, rows1)
                return inner

            lax.fori_loop(0, SB // 2, _pair, 0)
            return carry

        lax.fori_loop(0, nsb, _superchunk, 0)
        plsc.subcore_barrier()
        pltpu.sync_copy(acc.at[pl.ds(s * rpt, rpt)],
                        out_hbm.at[c, pl.ds(s * rpt, rpt)])

    return scatter_kernel


def kernel(x, edge_index, batch, W1, b1, W2, b2, W3, b3, W4, b4, W5, b5,
           Wlin, blin):
    n, d = x.shape
    e = edge_index.shape[1]
    nclasses = blin.shape[0]
    f32 = jnp.float32

    ept = -(-e // NW)
    ch = -(-(-(-ept // CS)) // SB) * SB
    epad = NW * ch * CS

    rpt = -(-n // (NS * 8)) * 8
    npad = NS * rpt
    pad_idx = n + 8

    src = edge_index[0]
    dst = edge_index[1]
    epad_fill = jnp.full((epad - e,), pad_idx, jnp.int32)
    src_r = jnp.concatenate([src, epad_fill]).reshape(NW, ch, CS)
    dst_r = jnp.concatenate([dst, epad_fill]).reshape(NW, ch, CS)
    x_pad = jnp.pad(x, ((0, npad - n), (0, 0)))
    zeros16 = jnp.zeros((rpt, 16), f32)
    zerosd = jnp.zeros((rpt, d), f32)
    batch_r = batch.reshape(1, n)
    b1r, b2r, b3r, b4r, b5r = (bb.reshape(1, d) for bb in (b1, b2, b3, b4, b5))
    blin_r = blin.reshape(1, nclasses)

    deg_kernel = _make_deg_kernel(npad, ch, rpt)
    scatter_kernel = _make_scatter_kernel(npad, d, ch, rpt)

    def first_body(x_ref, w_ref, degp_ref, y_out, dinv_out):
        deg = degp_ref[0, :, 0:1] + degp_ref[1, :, 0:1] + 1.0
        rows = lax.broadcasted_iota(jnp.int32, (npad, 1), 0)
        dinv = jnp.where(rows < n, lax.rsqrt(deg), 0.0)
        dinv_out[...] = dinv
        xw = jnp.dot(x_ref[...], w_ref[...], preferred_element_type=f32)
        y_out[...] = xw * dinv

    def mid_body(accp_ref, y_ref, dinv_ref, b_ref, w_ref, y_out):
        dinv = dinv_ref[...]
        pre = dinv * (accp_ref[0] + accp_ref[1] + y_ref[...]) + b_ref[...]
        h = jnp.maximum(pre, 0.0)
        y_out[...] = jnp.dot(h, w_ref[...], preferred_element_type=f32) * dinv

    def final_body(accp_ref, y_ref, dinv_ref, b_ref, batch_ref, wlin_ref,
                   blin_ref, out_ref):
        dinv = dinv_ref[...]
        h5 = dinv * (accp_ref[0] + accp_ref[1] + y_ref[...]) + b_ref[...]
        h = h5[:n]
        seg = (batch_ref[...] ==
               lax.broadcasted_iota(jnp.int32, (NGRAPHS, n), 0)).astype(f32)
        sums = jnp.dot(seg, h, preferred_element_type=f32)
        cnt = jnp.sum(seg, axis=1, keepdims=True)
        gmean = sums / jnp.maximum(cnt, 1.0)
        out_ref[...] = (jnp.dot(gmean, wlin_ref[...],
                                preferred_element_type=f32) + blin_ref[...])

    first_call = pl.pallas_call(first_body, out_shape=[
        jax.ShapeDtypeStruct((npad, d), f32),
        jax.ShapeDtypeStruct((npad, 1), f32),
    ])
    mid_call = pl.pallas_call(mid_body,
                              out_shape=jax.ShapeDtypeStruct((npad, d), f32))
    final_call = pl.pallas_call(
        final_body, out_shape=jax.ShapeDtypeStruct((NGRAPHS, nclasses), f32))

    degp = deg_kernel(dst_r, zeros16)
    y, dinv = first_call(x_pad, W1, degp)
    for w_next, b_cur in ((W2, b1r), (W3, b2r), (W4, b3r), (W5, b4r)):
        accp = scatter_kernel(y, src_r, dst_r, zerosd)
        y = mid_call(accp, y, dinv, b_cur, w_next)
    accp = scatter_kernel(y, src_r, dst_r, zerosd)
    return final_call(accp, y, dinv, b5r, batch_r, Wlin, blin_r)


# speedup vs baseline: 1.4735x; 1.4735x over previous
import functools

import jax
import jax.numpy as jnp
from jax import lax
from jax.experimental import pallas as pl
from jax.experimental.pallas import tpu as pltpu
from jax.experimental.pallas import tpu_sc as plsc

NC = 2
NS = 16
NW = NC * NS

CS = 128
SB = 16
NGRAPHS = 64


def _sc_mesh():
    return plsc.VectorSubcoreMesh(core_axis_name="c", subcore_axis_name="s")


def _make_deg_kernel(npad, ch, rpt):
    @functools.partial(
        pl.kernel,
        out_type=jax.ShapeDtypeStruct((NC, npad, 16), jnp.float32),
        mesh=_sc_mesh(),
        scratch_types=[
            pltpu.VMEM((ch, CS), jnp.int32),
            pltpu.VMEM((CS, 16), jnp.float32),
            pltpu.VMEM_SHARED((npad, 16), jnp.float32),
        ],
    )
    def deg_kernel(dst_hbm, zeros_hbm, out_hbm, dst_v, ones_v, dacc):
        c = lax.axis_index("c")
        s = lax.axis_index("s")
        wid = s * NC + c
        pltpu.sync_copy(dst_hbm.at[wid], dst_v)

        def _init(i, carry):
            ones_v[i, :] = jnp.ones((16,), jnp.float32)
            return carry

        lax.fori_loop(0, CS, _init, 0)
        pltpu.sync_copy(zeros_hbm, dacc.at[pl.ds(s * rpt, rpt)])
        plsc.subcore_barrier()

        def _body(j, carry):
            pltpu.sync_copy(ones_v, dacc.at[dst_v.at[j]], add=True)
            return carry

        lax.fori_loop(0, ch, _body, 0)
        plsc.subcore_barrier()
        pltpu.sync_copy(dacc.at[pl.ds(s * rpt, rpt)],
                        out_hbm.at[c, pl.ds(s * rpt, rpt)])

    return deg_kernel


def _make_scatter_kernel(npad, d, ch, rpt):
    @functools.partial(
        pl.kernel,
        out_type=jax.ShapeDtypeStruct((NC, npad, d), jnp.float32),
        mesh=_sc_mesh(),
        scratch_types=[
            pltpu.VMEM((ch, CS), jnp.int32),
            pltpu.VMEM((ch, CS), jnp.int32),
            pltpu.VMEM((CS, d), jnp.float32),
            pltpu.VMEM_SHARED((npad, d), jnp.float32),
            pltpu.SemaphoreType.DMA,
        ],
    )
    def scatter_kernel(y_hbm, src_hbm, dst_hbm, zeros_hbm, out_hbm,
                       src_v, dst_v, rows_v, acc, sem):
        c = lax.axis_index("c")
        s = lax.axis_index("s")
        wid = s * NC + c
        pltpu.sync_copy(src_hbm.at[wid], src_v)
        pltpu.sync_copy(dst_hbm.at[wid], dst_v)
        pltpu.sync_copy(zeros_hbm, acc.at[pl.ds(s * rpt, rpt)])
        plsc.subcore_barrier()

        def _body(j, carry):
            pltpu.async_copy(y_hbm.at[src_v.at[j]], rows_v, sem).wait()
            pltpu.sync_copy(rows_v, acc.at[dst_v.at[j]], add=True)
            return carry

        lax.fori_loop(0, ch, _body, 0)
        plsc.subcore_barrier()
        pltpu.sync_copy(acc.at[pl.ds(s * rpt, rpt)],
                        out_hbm.at[c, pl.ds(s * rpt, rpt)])

    return scatter_kernel


def kernel(x, edge_index, batch, W1, b1, W2, b2, W3, b3, W4, b4, W5, b5,
           Wlin, blin):
    n, d = x.shape
    e = edge_index.shape[1]
    nclasses = blin.shape[0]
    f32 = jnp.float32

    ept = -(-e // NW)
    ch = -(-ept // CS)
    epad = NW * ch * CS

    rpt = -(-n // (NS * 8)) * 8
    npad = NS * rpt
    pad_idx = n + 8

    src = edge_index[0]
    dst = edge_index[1]
    epad_fill = jnp.full((epad - e,), pad_idx, jnp.int32)
    src_r = jnp.concatenate([src, epad_fill]).reshape(NW, ch, CS)
    dst_r = jnp.concatenate([dst, epad_fill]).reshape(NW, ch, CS)
    x_pad = jnp.pad(x, ((0, npad - n), (0, 0)))
    zeros16 = jnp.zeros((rpt, 16), f32)
    zerosd = jnp.zeros((rpt, d), f32)
    batch_r = batch.reshape(1, n)
    b1r, b2r, b3r, b4r, b5r = (bb.reshape(1, d) for bb in (b1, b2, b3, b4, b5))
    blin_r = blin.reshape(1, nclasses)

    deg_kernel = _make_deg_kernel(npad, ch, rpt)
    scatter_kernel = _make_scatter_kernel(npad, d, ch, rpt)

    def first_body(x_ref, w_ref, degp_ref, y_out, dinv_out):
        deg = degp_ref[0, :, 0:1] + degp_ref[1, :, 0:1] + 1.0
        rows = lax.broadcasted_iota(jnp.int32, (npad, 1), 0)
        dinv = jnp.where(rows < n, lax.rsqrt(deg), 0.0)
        dinv_out[...] = dinv
        xw = jnp.dot(x_ref[...], w_ref[...], preferred_element_type=f32)
        y_out[...] = xw * dinv

    def mid_body(accp_ref, y_ref, dinv_ref, b_ref, w_ref, y_out):
        dinv = dinv_ref[...]
        pre = dinv * (accp_ref[0] + accp_ref[1] + y_ref[...]) + b_ref[...]
        h = jnp.maximum(pre, 0.0)
        y_out[...] = jnp.dot(h, w_ref[...], preferred_element_type=f32) * dinv

    def final_body(accp_ref, y_ref, dinv_ref, b_ref, batch_ref, wlin_ref,
                   blin_ref, out_ref):
        dinv = dinv_ref[...]
        h5 = dinv * (accp_ref[0] + accp_ref[1] + y_ref[...]) + b_ref[...]
        h = h5[:n]
        seg = (batch_ref[...] ==
               lax.broadcasted_iota(jnp.int32, (NGRAPHS, n), 0)).astype(f32)
        sums = jnp.dot(seg, h, preferred_element_type=f32)
        cnt = jnp.sum(seg, axis=1, keepdims=True)
        gmean = sums / jnp.maximum(cnt, 1.0)
        out_ref[...] = (jnp.dot(gmean, wlin_ref[...],
                                preferred_element_type=f32) + blin_ref[...])

    first_call = pl.pallas_call(first_body, out_shape=[
        jax.ShapeDtypeStruct((npad, d), f32),
        jax.ShapeDtypeStruct((npad, 1), f32),
    ])
    mid_call = pl.pallas_call(mid_body,
                              out_shape=jax.ShapeDtypeStruct((npad, d), f32))
    final_call = pl.pallas_call(
        final_body, out_shape=jax.ShapeDtypeStruct((NGRAPHS, nclasses), f32))

    degp = deg_kernel(dst_r, zeros16)
    y, dinv = first_call(x_pad, W1, degp)
    for w_next, b_cur in ((W2, b1r), (W3, b2r), (W4, b3r), (W5, b4r)):
        accp = scatter_kernel(y, src_r, dst_r, zerosd)
        y = mid_call(accp, y, dinv, b_cur, w_next)
    accp = scatter_kernel(y, src_r, dst_r, zerosd)
    return final_call(accp, y, dinv, b5r, batch_r, Wlin, blin_r)
